# baseline (speedup 1.0000x reference)
import functools
import jax
import jax.numpy as jnp
from jax import lax
from jax.experimental import pallas as pl
from jax.experimental.pallas import tpu as pltpu
from jax.experimental.pallas import tpu_sc as plsc

_B = 8
_D = 128
_H = 128
_W = 224
_CPW = 8
_L = 16


def _sc_kernel(col_hbm, row_hbm, out_hbm, tab, block, sem):
    wid = lax.axis_index("s") * 2 + lax.axis_index("c")
    is_col = wid < 16
    c0 = wid * _CPW

    @pl.when(is_col)
    def _():
        pltpu.sync_copy(col_hbm.at[0:_W], tab)

    @pl.when(jnp.logical_not(is_col))
    def _():
        pltpu.sync_copy(row_hbm.at[0:_H], tab.at[0:_H])

    lanes = lax.iota(jnp.int32, _L)
    nh = _H // _L

    def copies(par, c_abs):
        return [
            pltpu.make_async_copy(
                block.at[par],
                out_hbm.at[b, c_abs],
                sem.at[par],
            )
            for b in range(_B)
        ]

    for j in range(_CPW):
        par = j % 2
        c_abs = c0 + j

        if j >= 2:
            for cp in copies(par, c_abs):
                cp.wait()

        @pl.when(is_col)
        def _build_col():
            def body(w, _):
                vec = plsc.load_gather(
                    tab,
                    [jnp.full((_L,), w, jnp.int32),
                     jnp.full((_L,), c_abs, jnp.int32)],
                )
                for k in range(nh):
                    block[par, w, pl.ds(k * _L, _L)] = vec
                return 0

            lax.fori_loop(0, _W, body, 0, unroll=False)

        @pl.when(jnp.logical_not(is_col))
        def _build_row():
            for k in range(nh):
                vec = plsc.load_gather(
                    tab,
                    [lanes + (k * _L), jnp.full((_L,), c_abs - _D, jnp.int32)],
                )
                block[par, 0, pl.ds(k * _L, _L)] = vec

            def body(w, _):
                for k in range(nh):
                    block[par, w, pl.ds(k * _L, _L)] = block[
                        par, 0, pl.ds(k * _L, _L)
                    ]
                return 0

            lax.fori_loop(1, _W, body, 0, unroll=False)

        for cp in copies(par, c_abs):
            cp.start()

    for cp in copies(0, c0):
        cp.wait()
    for cp in copies(1, c0):
        cp.wait()


def kernel(x, row_embed, col_embed):
    B, C, H, W = x.shape

    mesh = plsc.VectorSubcoreMesh(core_axis_name="c", subcore_axis_name="s")
    k = functools.partial(
        pl.kernel,
        mesh=mesh,
        out_type=jax.ShapeDtypeStruct((B, C, W, H), x.dtype),
        scratch_types=[
            pltpu.VMEM((_W, _D), jnp.float32),
            pltpu.VMEM((2, _W, _H), jnp.float32),
            pltpu.SemaphoreType.DMA((2,)),
        ],
        compiler_params=pltpu.CompilerParams(needs_layout_passes=False),
    )(_sc_kernel)
    out = k(col_embed, row_embed)
    return jnp.transpose(out, (0, 1, 3, 2))


# speedup vs baseline: 1.2242x; 1.2242x over previous
import jax
import jax.numpy as jnp
from jax.experimental import pallas as pl
from jax.experimental.pallas import tpu as pltpu

_B = 8


def _pos_kernel(col_ref, row_ref, out_ref, scratch, sem):
    s = pl.program_id(0)
    d, w, h = scratch.shape[1], scratch.shape[2], scratch.shape[3]

    @pl.when(s == 0)
    def _col():
        colT = col_ref[...].T
        scratch[0] = jnp.broadcast_to(colT[:, :, None], (d, w, h))

    @pl.when(s == 1)
    def _row():
        rowT = row_ref[...].T
        scratch[1] = jnp.broadcast_to(rowT[:, None, :], (d, w, h))

    def copies(ss):
        return [
            pltpu.make_async_copy(
                scratch.at[ss],
                out_ref.at[b, pl.ds(ss * d, d), :, :],
                sem.at[ss],
            )
            for b in range(_B)
        ]

    for c in copies(s):
        c.start()

    @pl.when(s == 1)
    def _drain():
        for c in copies(1):
            c.wait()
        for c in copies(0):
            c.wait()


def kernel(x, row_embed, col_embed):
    B, C, H, W = x.shape
    d = col_embed.shape[1]

    col = col_embed[:W]
    row = row_embed[:H]

    out = pl.pallas_call(
        _pos_kernel,
        grid=(2,),
        in_specs=[
            pl.BlockSpec((W, d), lambda s: (0, 0)),
            pl.BlockSpec((H, d), lambda s: (0, 0)),
        ],
        out_specs=pl.BlockSpec(memory_space=pltpu.MemorySpace.HBM),
        out_shape=jax.ShapeDtypeStruct((B, C, W, H), x.dtype),
        scratch_shapes=[
            pltpu.VMEM((2, d, W, H), jnp.float32),
            pltpu.SemaphoreType.DMA((2,)),
        ],
    )(col, row)
    return jnp.transpose(out, (0, 1, 3, 2))
